# Initial kernel scaffold; baseline (speedup 1.0000x reference)
import jax
import jax.numpy as jnp
from jax.experimental import pallas as pl


def kernel(input, adj, W, a):
    raise NotImplementedError("write your pallas kernel here")


# speedup vs baseline: 477.0343x; 477.0343x over previous
import functools

import jax
import jax.numpy as jnp
from jax.experimental import pallas as pl
from jax.experimental.pallas import tpu as pltpu

IN_F = 128
OUT_F = 64
ALPHA = 0.2
N_NODES = 1024
BI = 256


def _gat_block(x_ref, adj_ref, w_ref, a_ref, out_ref):
    i = pl.program_id(0)
    x = x_ref[...]
    w1 = w_ref[:IN_F, :]
    w2 = w_ref[IN_F:, :]
    a = a_ref[...]

    whi = jnp.dot(x, w1, preferred_element_type=jnp.float32)
    vt = jax.lax.dot_general(w2, x, (((0,), (1,)), ((), ())),
                             preferred_element_type=jnp.float32)
    xi = x_ref[pl.ds(i * BI, BI), :]
    whi_blk = jnp.dot(xi, w1, preferred_element_type=jnp.float32)

    p = jnp.dot(whi_blk, a.T, preferred_element_type=jnp.float32)
    q = jnp.dot(a, vt, preferred_element_type=jnp.float32)
    e = ALPHA * (p + q)

    av = (1.0 - ALPHA) * a
    for k in range(OUT_F):
        z = whi_blk[:, k:k + 1] + vt[k:k + 1, :]
        e = e + av[0, k] * jnp.maximum(z, 0.0)

    mask = adj_ref[...] != 0.0
    neg_inf = jnp.float32(-jnp.inf)
    m = jnp.max(jnp.where(mask, e, neg_inf), axis=1, keepdims=True)
    ew = jnp.where(mask, jnp.exp(e - m), 0.0)
    rowsum = jnp.sum(ew, axis=1, keepdims=True)

    hp = jnp.dot(ew, whi, preferred_element_type=jnp.float32)
    hp = hp / rowsum
    out_ref[...] = jnp.where(hp > 0.0, hp, jnp.exp(hp) - 1.0)


@jax.jit
def kernel(input, adj, W, a):
    n = input.shape[0]
    grid = n // BI
    return pl.pallas_call(
        _gat_block,
        grid=(grid,),
        in_specs=[
            pl.BlockSpec((n, IN_F), lambda i: (0, 0)),
            pl.BlockSpec((BI, n), lambda i: (i, 0)),
            pl.BlockSpec((2 * IN_F, OUT_F), lambda i: (0, 0)),
            pl.BlockSpec((1, OUT_F), lambda i: (0, 0)),
        ],
        out_specs=pl.BlockSpec((BI, OUT_F), lambda i: (i, 0)),
        out_shape=jax.ShapeDtypeStruct((n, OUT_F), jnp.float32),
    )(input, adj, W, a)
